# baseline (speedup 1.0000x reference)
import functools

import jax
import jax.numpy as jnp
from jax import lax
from jax.experimental import pallas as pl
from jax.experimental.pallas import tpu as pltpu
from jax.experimental.pallas import tpu_sc as plsc

NI, NJ, NK, D = 1024, 256, 256, 128
E = 32768
NC, NS, LANES = 2, 16, 16
NW = NC * NS
EPW = E // NW
ROWS = EPW // 128
ZCH = 2048
SL_W = NI * NJ // NS


def _hist_body(si_hbm, sj_hbm, sk_hbm, w_hbm,
               wlr_o, wlu_o,
               si_v, sj_v, sk_v, w_v, ilr_v, ilu_v, zero_v,
               b1_v, b2_v, sem_a, sem_b,
               wlr_s, wlu_s):
    c = lax.axis_index("c")
    s = lax.axis_index("s")
    wid = s * NC + c

    h_in = [
        pltpu.async_copy(si_hbm.at[wid], si_v, sem_a),
        pltpu.async_copy(sj_hbm.at[wid], sj_v, sem_a),
        pltpu.async_copy(sk_hbm.at[wid], sk_v, sem_a),
        pltpu.async_copy(w_hbm.at[wid], w_v, sem_a),
    ]

    def _zb(t, _):
        zero_v[pl.ds(t * LANES, LANES)] = jnp.zeros((LANES,), jnp.float32)
        return 0
    lax.fori_loop(0, ZCH // LANES, _zb, 0)

    h_z = []
    for t in range(SL_W // ZCH):
        h_z.append(pltpu.async_copy(zero_v, wlr_s.at[pl.ds(s * SL_W + t * ZCH, ZCH)], sem_b))
        h_z.append(pltpu.async_copy(zero_v, wlu_s.at[pl.ds(s * SL_W + t * ZCH, ZCH)], sem_b))

    for h in h_in:
        h.wait()

    for r in range(ROWS):
        for q in range(128 // LANES):
            sl = pl.ds(q * LANES, LANES)
            siv = si_v[r, sl] << 7
            sjv = sj_v[r, sl]
            skv = sk_v[r, sl]
            ilr_v[r, sl] = ((sjv >> 7) << 17) + siv + (sjv & 127)
            ilu_v[r, sl] = ((skv >> 7) << 17) + siv + (skv & 127)

    for h in h_z:
        h.wait()
    plsc.subcore_barrier()

    h_sc = []
    for r in range(ROWS):
        h_sc.append(pltpu.async_copy(w_v.at[r], wlr_s.at[ilr_v.at[r]], sem_a, add=True))
        h_sc.append(pltpu.async_copy(w_v.at[r], wlu_s.at[ilu_v.at[r]], sem_b, add=True))
    for h in h_sc:
        h.wait()

    plsc.subcore_barrier()

    l1 = [
        pltpu.async_copy(wlr_s.at[pl.ds(s * SL_W, SL_W)], b1_v, sem_a),
        pltpu.async_copy(wlu_s.at[pl.ds(s * SL_W, SL_W)], b2_v, sem_b),
    ]
    for h in l1:
        h.wait()
    base = c * (NI * NJ) + s * SL_W
    l2 = [
        pltpu.async_copy(b1_v, wlr_o.at[pl.ds(base, SL_W)], sem_a),
        pltpu.async_copy(b2_v, wlu_o.at[pl.ds(base, SL_W)], sem_b),
    ]
    for h in l2:
        h.wait()


def _hist_call():
    return functools.partial(
        pl.kernel,
        out_type=(
            jax.ShapeDtypeStruct((NC * NI * NJ,), jnp.float32),
            jax.ShapeDtypeStruct((NC * NI * NK,), jnp.float32),
        ),
        mesh=plsc.VectorSubcoreMesh(core_axis_name="c", subcore_axis_name="s"),
        scratch_types=[
            pltpu.VMEM((ROWS, 128), jnp.int32),
            pltpu.VMEM((ROWS, 128), jnp.int32),
            pltpu.VMEM((ROWS, 128), jnp.int32),
            pltpu.VMEM((ROWS, 128), jnp.float32),
            pltpu.VMEM((ROWS, 128), jnp.int32),
            pltpu.VMEM((ROWS, 128), jnp.int32),
            pltpu.VMEM((ZCH,), jnp.float32),
            pltpu.VMEM((SL_W,), jnp.float32),
            pltpu.VMEM((SL_W,), jnp.float32),
            pltpu.SemaphoreType.DMA,
            pltpu.SemaphoreType.DMA,
            pltpu.VMEM_SHARED((NI * NJ,), jnp.float32),
            pltpu.VMEM_SHARED((NI * NK,), jnp.float32),
        ],
    )


def _tables_body(l_ref, r_ref, u_ref, nu_ref, tau_ref,
                 erl_ref, eul_ref, dqlr_ref, dqlu_ref):
    f32 = jnp.float32
    Lm = l_ref[:]
    ones_row = jnp.ones((1, D), f32)
    eps = f32(1e-6)
    deps2 = f32(D * 1e-12)
    na = jnp.sum(Lm * Lm, axis=1, keepdims=True)
    sa = jnp.sum(Lm, axis=1, keepdims=True)

    def tables(Bm):
        G = lax.dot_general(Lm, Bm, (((1,), (1,)), ((), ())),
                            preferred_element_type=f32,
                            precision=lax.Precision.HIGHEST)
        nb = lax.dot_general(ones_row, Bm * Bm, (((1,), (1,)), ((), ())),
                             preferred_element_type=f32,
                             precision=lax.Precision.HIGHEST)
        sb = lax.dot_general(ones_row, Bm, (((1,), (1,)), ((), ())),
                             preferred_element_type=f32,
                             precision=lax.Precision.HIGHEST)
        base = na + nb - 2.0 * G
        sdiff = sb - sa
        d_nl = jnp.sqrt(jnp.maximum(base + 2.0 * eps * sdiff + deps2, 0.0)) + eps
        d_q = jnp.sqrt(jnp.maximum(base - 2.0 * eps * sdiff + deps2, 0.0))
        return d_nl, d_q

    d_rl, dq_lr = tables(r_ref[:])
    d_ul, dq_lu = tables(u_ref[:])
    erl_ref[...] = jnp.exp(nu_ref[:] - d_rl).astype(jnp.bfloat16)
    eul_ref[...] = jnp.exp(tau_ref[:] - d_ul).astype(jnp.bfloat16)
    dqlr_ref[...] = (nu_ref[:] - dq_lr).astype(jnp.bfloat16)
    dqlu_ref[...] = (tau_ref[:] - dq_lu).astype(jnp.bfloat16)


def _combine_body(rho_ref, erl_ref, eul_ref, dqlr_ref, dqlu_ref,
                  wlr_ref, wlu_ref, out_ref):
    f32 = jnp.float32
    rho_c = rho_ref[:]

    def planes(wref):
        h0 = wref[0:NI, :] + wref[2 * NI:3 * NI, :]
        h1 = wref[NI:2 * NI, :] + wref[3 * NI:4 * NI, :]
        return h0, h1

    wlr0, wlr1 = planes(wlr_ref)
    wlu0, wlu1 = planes(wlu_ref)

    cnt_i = (jnp.sum(wlr0, axis=1, keepdims=True)
             + jnp.sum(wlr1, axis=1, keepdims=True))
    pres_i = (cnt_i > 0.0).astype(f32)
    n_i = jnp.sum(pres_i)
    iota_i = lax.broadcasted_iota(jnp.int32, (NI, 1), 0)
    min_i = jnp.min(jnp.where(cnt_i > 0.0, iota_i, jnp.int32(NI)))
    mi = pres_i + jnp.where(iota_i == min_i, f32(NI) - n_i, 0.0)

    def mult_cols(w0, w1, N):
        cs0 = jnp.sum(w0, axis=0, keepdims=True)
        cs1 = jnp.sum(w1, axis=0, keepdims=True)
        p0 = (cs0 > 0.0).astype(f32)
        p1 = (cs1 > 0.0).astype(f32)
        n = jnp.sum(p0) + jnp.sum(p1)
        io0 = lax.broadcasted_iota(jnp.int32, (1, 128), 1)
        io1 = io0 + 128
        mn = jnp.minimum(jnp.min(jnp.where(cs0 > 0.0, io0, jnp.int32(N))),
                         jnp.min(jnp.where(cs1 > 0.0, io1, jnp.int32(N))))
        extra = f32(N) - n
        m0 = p0 + jnp.where(io0 == mn, extra, 0.0)
        m1 = p1 + jnp.where(io1 == mn, extra, 0.0)
        return m0, m1

    mj0, mj1 = mult_cols(wlr0, wlr1, NJ)
    mk0, mk1 = mult_cols(wlu0, wlu1, NK)

    def wsum(eref, m0, m1):
        e0 = eref[:, 0:128].astype(f32)
        e1 = eref[:, 128:256].astype(f32)
        d = (((1,), (1,)), ((), ()))
        return (lax.dot_general(e0, m0, d, preferred_element_type=f32,
                                precision=lax.Precision.HIGHEST)
                + lax.dot_general(e1, m1, d, preferred_element_type=f32,
                                  precision=lax.Precision.HIGHEST))

    Acol = wsum(erl_ref, mj0, mj1)
    Bcol = jnp.exp(rho_c) * wsum(eul_ref, mk0, mk1)
    z1 = jnp.sum(mi * Acol * Bcol)

    phi_lr = dqlr_ref[:].astype(f32)
    phi_lu = dqlu_ref[:].astype(f32)
    z2 = (jnp.sum(wlr0 * phi_lr[:, 0:128])
          + jnp.sum(wlr1 * phi_lr[:, 128:256])
          + jnp.sum(wlu0 * (rho_c + phi_lu[:, 0:128]))
          + jnp.sum(wlu1 * (rho_c + phi_lu[:, 128:256])))

    out_ref[...] = jnp.broadcast_to(z2 - z1, (1, 1))


def kernel(latent_l, latent_r, latent_u, rho, nu, tau,
           sample_i, sample_j, sample_k, sample_weights):
    si = sample_i.astype(jnp.int32).reshape(NW, ROWS, 128)
    sj = sample_j.astype(jnp.int32).reshape(NW, ROWS, 128)
    sk = sample_k.astype(jnp.int32).reshape(NW, ROWS, 128)
    w = (sample_weights + jnp.float32(1e-20)).reshape(NW, ROWS, 128)

    wlr2, wlu2 = _hist_call()(_hist_body)(si, sj, sk, w)

    nu_row = nu[:NJ].reshape(1, NJ)
    tau_row = tau[:NK].reshape(1, NK)
    erl, eul, dqlr, dqlu = pl.pallas_call(
        _tables_body,
        out_shape=(jax.ShapeDtypeStruct((NI, NJ), jnp.bfloat16),
                   jax.ShapeDtypeStruct((NI, NK), jnp.bfloat16),
                   jax.ShapeDtypeStruct((NI, NJ), jnp.bfloat16),
                   jax.ShapeDtypeStruct((NI, NK), jnp.bfloat16)),
        grid=(1,),
        in_specs=[
            pl.BlockSpec((NI, D), lambda i: (0, 0)),
            pl.BlockSpec((NJ, D), lambda i: (0, 0)),
            pl.BlockSpec((NK, D), lambda i: (0, 0)),
            pl.BlockSpec((1, NJ), lambda i: (0, 0)),
            pl.BlockSpec((1, NK), lambda i: (0, 0)),
        ],
        out_specs=(pl.BlockSpec((NI, NJ), lambda i: (0, 0)),
                   pl.BlockSpec((NI, NK), lambda i: (0, 0)),
                   pl.BlockSpec((NI, NJ), lambda i: (0, 0)),
                   pl.BlockSpec((NI, NK), lambda i: (0, 0))),
    )(latent_l, latent_r, latent_u, nu_row, tau_row)

    out = pl.pallas_call(
        _combine_body,
        out_shape=jax.ShapeDtypeStruct((1, 1), jnp.float32),
    )(rho[:NI].reshape(NI, 1), erl, eul, dqlr, dqlu,
      wlr2.reshape(NC * 2 * NI, 128), wlu2.reshape(NC * 2 * NI, 128))
    return out[0, 0]


# speedup vs baseline: 1.0306x; 1.0306x over previous
import functools

import jax
import jax.numpy as jnp
from jax import lax
from jax.experimental import pallas as pl
from jax.experimental.pallas import tpu as pltpu
from jax.experimental.pallas import tpu_sc as plsc

NI, NJ, NK, D = 1024, 256, 256, 128
E = 32768
NC, NS, LANES = 2, 16, 16
NW = NC * NS
EPW = E // NW
ROWS = EPW // 128
ZCH = 2048
SL_W = NI * NJ // NS


def _hist_body(si_hbm, sj_hbm, sk_hbm, w_hbm,
               wlr_o, wlu_o,
               si_v, sj_v, sk_v, w_v, ilr_v, ilu_v, zero_v,
               b1_v, b2_v, sem_a, sem_b,
               wlr_s, wlu_s):
    c = lax.axis_index("c")
    s = lax.axis_index("s")
    wid = s * NC + c

    h_in = [
        pltpu.async_copy(si_hbm.at[wid], si_v, sem_a),
        pltpu.async_copy(sj_hbm.at[wid], sj_v, sem_a),
        pltpu.async_copy(sk_hbm.at[wid], sk_v, sem_a),
        pltpu.async_copy(w_hbm.at[wid], w_v, sem_a),
    ]

    def _zb(t, _):
        zero_v[pl.ds(t * LANES, LANES)] = jnp.zeros((LANES,), jnp.float32)
        return 0
    lax.fori_loop(0, ZCH // LANES, _zb, 0)

    h_z = []
    for t in range(SL_W // ZCH):
        h_z.append(pltpu.async_copy(zero_v, wlr_s.at[pl.ds(s * SL_W + t * ZCH, ZCH)], sem_b))
        h_z.append(pltpu.async_copy(zero_v, wlu_s.at[pl.ds(s * SL_W + t * ZCH, ZCH)], sem_b))

    for h in h_in:
        h.wait()

    for r in range(ROWS):
        for q in range(128 // LANES):
            sl = pl.ds(q * LANES, LANES)
            siv = si_v[r, sl] << 7
            sjv = sj_v[r, sl]
            skv = sk_v[r, sl]
            ilr_v[r, sl] = ((sjv >> 7) << 17) + siv + (sjv & 127)
            ilu_v[r, sl] = ((skv >> 7) << 17) + siv + (skv & 127)
            w_v[r, sl] = w_v[r, sl] + jnp.float32(1e-20)

    for h in h_z:
        h.wait()
    plsc.subcore_barrier()

    h_sc = []
    for r in range(ROWS):
        h_sc.append(pltpu.async_copy(w_v.at[r], wlr_s.at[ilr_v.at[r]], sem_a, add=True))
        h_sc.append(pltpu.async_copy(w_v.at[r], wlu_s.at[ilu_v.at[r]], sem_b, add=True))
    for h in h_sc:
        h.wait()

    plsc.subcore_barrier()

    l1 = [
        pltpu.async_copy(wlr_s.at[pl.ds(s * SL_W, SL_W)], b1_v, sem_a),
        pltpu.async_copy(wlu_s.at[pl.ds(s * SL_W, SL_W)], b2_v, sem_b),
    ]
    for h in l1:
        h.wait()
    base = c * (NI * NJ) + s * SL_W
    l2 = [
        pltpu.async_copy(b1_v, wlr_o.at[pl.ds(base, SL_W)], sem_a),
        pltpu.async_copy(b2_v, wlu_o.at[pl.ds(base, SL_W)], sem_b),
    ]
    for h in l2:
        h.wait()


def _hist_call():
    return functools.partial(
        pl.kernel,
        out_type=(
            jax.ShapeDtypeStruct((NC * NI * NJ,), jnp.float32),
            jax.ShapeDtypeStruct((NC * NI * NK,), jnp.float32),
        ),
        mesh=plsc.VectorSubcoreMesh(core_axis_name="c", subcore_axis_name="s"),
        scratch_types=[
            pltpu.VMEM((ROWS, 128), jnp.int32),
            pltpu.VMEM((ROWS, 128), jnp.int32),
            pltpu.VMEM((ROWS, 128), jnp.int32),
            pltpu.VMEM((ROWS, 128), jnp.float32),
            pltpu.VMEM((ROWS, 128), jnp.int32),
            pltpu.VMEM((ROWS, 128), jnp.int32),
            pltpu.VMEM((ZCH,), jnp.float32),
            pltpu.VMEM((SL_W,), jnp.float32),
            pltpu.VMEM((SL_W,), jnp.float32),
            pltpu.SemaphoreType.DMA,
            pltpu.SemaphoreType.DMA,
            pltpu.VMEM_SHARED((NI * NJ,), jnp.float32),
            pltpu.VMEM_SHARED((NI * NK,), jnp.float32),
        ],
    )


def _tables_body(l_ref, r_ref, u_ref, nu_ref, tau_ref,
                 erl_ref, eul_ref, dqlr_ref, dqlu_ref):
    f32 = jnp.float32
    Lm = l_ref[:]
    ones_row = jnp.ones((1, D), f32)
    eps = f32(1e-6)
    deps2 = f32(D * 1e-12)
    na = jnp.sum(Lm * Lm, axis=1, keepdims=True)
    sa = jnp.sum(Lm, axis=1, keepdims=True)

    def tables(Bm):
        G = lax.dot_general(Lm, Bm, (((1,), (1,)), ((), ())),
                            preferred_element_type=f32,
                            precision=lax.Precision.HIGHEST)
        nb = lax.dot_general(ones_row, Bm * Bm, (((1,), (1,)), ((), ())),
                             preferred_element_type=f32,
                             precision=lax.Precision.HIGHEST)
        sb = lax.dot_general(ones_row, Bm, (((1,), (1,)), ((), ())),
                             preferred_element_type=f32,
                             precision=lax.Precision.HIGHEST)
        base = na + nb - 2.0 * G
        sdiff = sb - sa
        d_nl = jnp.sqrt(jnp.maximum(base + 2.0 * eps * sdiff + deps2, 0.0)) + eps
        d_q = jnp.sqrt(jnp.maximum(base - 2.0 * eps * sdiff + deps2, 0.0))
        return d_nl, d_q

    d_rl, dq_lr = tables(r_ref[:])
    d_ul, dq_lu = tables(u_ref[:])
    erl_ref[...] = jnp.exp(nu_ref[:] - d_rl).astype(jnp.bfloat16)
    eul_ref[...] = jnp.exp(tau_ref[:] - d_ul).astype(jnp.bfloat16)
    dqlr_ref[...] = (nu_ref[:] - dq_lr).astype(jnp.bfloat16)
    dqlu_ref[...] = (tau_ref[:] - dq_lu).astype(jnp.bfloat16)


def _combine_body(rho_ref, erl_ref, eul_ref, dqlr_ref, dqlu_ref,
                  wlr_ref, wlu_ref, out_ref):
    f32 = jnp.float32
    rho_c = rho_ref[:]

    def planes(wref):
        h0 = wref[0:NI, :] + wref[2 * NI:3 * NI, :]
        h1 = wref[NI:2 * NI, :] + wref[3 * NI:4 * NI, :]
        return h0, h1

    wlr0, wlr1 = planes(wlr_ref)
    wlu0, wlu1 = planes(wlu_ref)

    cnt_i = (jnp.sum(wlr0, axis=1, keepdims=True)
             + jnp.sum(wlr1, axis=1, keepdims=True))
    pres_i = (cnt_i > 0.0).astype(f32)
    n_i = jnp.sum(pres_i)
    iota_i = lax.broadcasted_iota(jnp.int32, (NI, 1), 0)
    min_i = jnp.min(jnp.where(cnt_i > 0.0, iota_i, jnp.int32(NI)))
    mi = pres_i + jnp.where(iota_i == min_i, f32(NI) - n_i, 0.0)

    def mult_cols(w0, w1, N):
        cs0 = jnp.sum(w0, axis=0, keepdims=True)
        cs1 = jnp.sum(w1, axis=0, keepdims=True)
        p0 = (cs0 > 0.0).astype(f32)
        p1 = (cs1 > 0.0).astype(f32)
        n = jnp.sum(p0) + jnp.sum(p1)
        io0 = lax.broadcasted_iota(jnp.int32, (1, 128), 1)
        io1 = io0 + 128
        mn = jnp.minimum(jnp.min(jnp.where(cs0 > 0.0, io0, jnp.int32(N))),
                         jnp.min(jnp.where(cs1 > 0.0, io1, jnp.int32(N))))
        extra = f32(N) - n
        m0 = p0 + jnp.where(io0 == mn, extra, 0.0)
        m1 = p1 + jnp.where(io1 == mn, extra, 0.0)
        return m0, m1

    mj0, mj1 = mult_cols(wlr0, wlr1, NJ)
    mk0, mk1 = mult_cols(wlu0, wlu1, NK)

    def wsum(eref, m0, m1):
        e0 = eref[:, 0:128].astype(f32)
        e1 = eref[:, 128:256].astype(f32)
        d = (((1,), (1,)), ((), ()))
        return (lax.dot_general(e0, m0, d, preferred_element_type=f32,
                                precision=lax.Precision.HIGHEST)
                + lax.dot_general(e1, m1, d, preferred_element_type=f32,
                                  precision=lax.Precision.HIGHEST))

    Acol = wsum(erl_ref, mj0, mj1)
    Bcol = jnp.exp(rho_c) * wsum(eul_ref, mk0, mk1)
    z1 = jnp.sum(mi * Acol * Bcol)

    phi_lr = dqlr_ref[:].astype(f32)
    phi_lu = dqlu_ref[:].astype(f32)
    z2 = (jnp.sum(wlr0 * phi_lr[:, 0:128])
          + jnp.sum(wlr1 * phi_lr[:, 128:256])
          + jnp.sum(wlu0 * (rho_c + phi_lu[:, 0:128]))
          + jnp.sum(wlu1 * (rho_c + phi_lu[:, 128:256])))

    out_ref[...] = jnp.broadcast_to(z2 - z1, (1, 1))


def kernel(latent_l, latent_r, latent_u, rho, nu, tau,
           sample_i, sample_j, sample_k, sample_weights):
    si = sample_i.astype(jnp.int32).reshape(NW, ROWS, 128)
    sj = sample_j.astype(jnp.int32).reshape(NW, ROWS, 128)
    sk = sample_k.astype(jnp.int32).reshape(NW, ROWS, 128)
    w = sample_weights.reshape(NW, ROWS, 128)

    wlr2, wlu2 = _hist_call()(_hist_body)(si, sj, sk, w)

    nu_row = nu[:NJ].reshape(1, NJ)
    tau_row = tau[:NK].reshape(1, NK)
    erl, eul, dqlr, dqlu = pl.pallas_call(
        _tables_body,
        out_shape=(jax.ShapeDtypeStruct((NI, NJ), jnp.bfloat16),
                   jax.ShapeDtypeStruct((NI, NK), jnp.bfloat16),
                   jax.ShapeDtypeStruct((NI, NJ), jnp.bfloat16),
                   jax.ShapeDtypeStruct((NI, NK), jnp.bfloat16)),
        grid=(1,),
        in_specs=[
            pl.BlockSpec((NI, D), lambda i: (0, 0)),
            pl.BlockSpec((NJ, D), lambda i: (0, 0)),
            pl.BlockSpec((NK, D), lambda i: (0, 0)),
            pl.BlockSpec((1, NJ), lambda i: (0, 0)),
            pl.BlockSpec((1, NK), lambda i: (0, 0)),
        ],
        out_specs=(pl.BlockSpec((NI, NJ), lambda i: (0, 0)),
                   pl.BlockSpec((NI, NK), lambda i: (0, 0)),
                   pl.BlockSpec((NI, NJ), lambda i: (0, 0)),
                   pl.BlockSpec((NI, NK), lambda i: (0, 0))),
    )(latent_l, latent_r, latent_u, nu_row, tau_row)

    out = pl.pallas_call(
        _combine_body,
        out_shape=jax.ShapeDtypeStruct((1, 1), jnp.float32),
    )(rho[:NI].reshape(NI, 1), erl, eul, dqlr, dqlu,
      wlr2.reshape(NC * 2 * NI, 128), wlu2.reshape(NC * 2 * NI, 128))
    return out[0, 0]


# speedup vs baseline: 1.0317x; 1.0010x over previous
import functools

import jax
import jax.numpy as jnp
from jax import lax
from jax.experimental import pallas as pl
from jax.experimental.pallas import tpu as pltpu
from jax.experimental.pallas import tpu_sc as plsc

NI, NJ, NK, D = 1024, 256, 256, 128
E = 32768
NC, NS, LANES = 2, 16, 16
NW = NC * NS
EPW = E // NW
ROWS = EPW // 128
ZCH = 2048
SL_W = NI * NJ // NS


def _hist_body(si_hbm, sj_hbm, sk_hbm, w_hbm,
               wlr_o, wlu_o,
               si_v, sj_v, sk_v, w_v, ilr_v, ilu_v, zero_v,
               b1_v, b2_v, sem_a, sem_b,
               wlr_s, wlu_s):
    c = lax.axis_index("c")
    s = lax.axis_index("s")
    wid = s * NC + c

    h_in = [
        pltpu.async_copy(si_hbm.at[wid], si_v, sem_a),
        pltpu.async_copy(sj_hbm.at[wid], sj_v, sem_a),
        pltpu.async_copy(sk_hbm.at[wid], sk_v, sem_a),
        pltpu.async_copy(w_hbm.at[wid], w_v, sem_a),
    ]

    def _zb(t, _):
        zero_v[pl.ds(t * LANES, LANES)] = jnp.zeros((LANES,), jnp.float32)
        return 0
    lax.fori_loop(0, ZCH // LANES, _zb, 0)

    h_z = []
    for t in range(SL_W // ZCH):
        h_z.append(pltpu.async_copy(zero_v, wlr_s.at[pl.ds(s * SL_W + t * ZCH, ZCH)], sem_b))
        h_z.append(pltpu.async_copy(zero_v, wlu_s.at[pl.ds(s * SL_W + t * ZCH, ZCH)], sem_b))

    for h in h_in:
        h.wait()

    for r in range(ROWS):
        for q in range(128 // LANES):
            sl = pl.ds(q * LANES, LANES)
            siv = si_v[r, sl] << 7
            sjv = sj_v[r, sl]
            skv = sk_v[r, sl]
            ilr_v[r, sl] = ((sjv >> 7) << 17) + siv + (sjv & 127)
            ilu_v[r, sl] = ((skv >> 7) << 17) + siv + (skv & 127)
            w_v[r, sl] = w_v[r, sl] + jnp.float32(1e-20)

    for h in h_z:
        h.wait()
    plsc.subcore_barrier()

    h_sc = []
    for r in range(ROWS):
        h_sc.append(pltpu.async_copy(w_v.at[r], wlr_s.at[ilr_v.at[r]], sem_a, add=True))
        h_sc.append(pltpu.async_copy(w_v.at[r], wlu_s.at[ilu_v.at[r]], sem_b, add=True))
    for h in h_sc:
        h.wait()

    plsc.subcore_barrier()

    base = c * (NI * NJ) + s * SL_W
    l2 = [
        pltpu.async_copy(wlr_s.at[pl.ds(s * SL_W, SL_W)], wlr_o.at[pl.ds(base, SL_W)], sem_a),
        pltpu.async_copy(wlu_s.at[pl.ds(s * SL_W, SL_W)], wlu_o.at[pl.ds(base, SL_W)], sem_b),
    ]
    for h in l2:
        h.wait()


def _hist_call():
    return functools.partial(
        pl.kernel,
        out_type=(
            jax.ShapeDtypeStruct((NC * NI * NJ,), jnp.float32),
            jax.ShapeDtypeStruct((NC * NI * NK,), jnp.float32),
        ),
        mesh=plsc.VectorSubcoreMesh(core_axis_name="c", subcore_axis_name="s"),
        scratch_types=[
            pltpu.VMEM((ROWS, 128), jnp.int32),
            pltpu.VMEM((ROWS, 128), jnp.int32),
            pltpu.VMEM((ROWS, 128), jnp.int32),
            pltpu.VMEM((ROWS, 128), jnp.float32),
            pltpu.VMEM((ROWS, 128), jnp.int32),
            pltpu.VMEM((ROWS, 128), jnp.int32),
            pltpu.VMEM((ZCH,), jnp.float32),
            pltpu.VMEM((SL_W,), jnp.float32),
            pltpu.VMEM((SL_W,), jnp.float32),
            pltpu.SemaphoreType.DMA,
            pltpu.SemaphoreType.DMA,
            pltpu.VMEM_SHARED((NI * NJ,), jnp.float32),
            pltpu.VMEM_SHARED((NI * NK,), jnp.float32),
        ],
    )


def _tables_body(l_ref, r_ref, u_ref, nu_ref, tau_ref,
                 erl_ref, eul_ref, dqlr_ref, dqlu_ref):
    f32 = jnp.float32
    Lm = l_ref[:]
    ones_row = jnp.ones((1, D), f32)
    eps = f32(1e-6)
    deps2 = f32(D * 1e-12)
    na = jnp.sum(Lm * Lm, axis=1, keepdims=True)
    sa = jnp.sum(Lm, axis=1, keepdims=True)

    def tables(Bm):
        G = lax.dot_general(Lm, Bm, (((1,), (1,)), ((), ())),
                            preferred_element_type=f32,
                            precision=lax.Precision.HIGHEST)
        nb = lax.dot_general(ones_row, Bm * Bm, (((1,), (1,)), ((), ())),
                             preferred_element_type=f32,
                             precision=lax.Precision.HIGHEST)
        sb = lax.dot_general(ones_row, Bm, (((1,), (1,)), ((), ())),
                             preferred_element_type=f32,
                             precision=lax.Precision.HIGHEST)
        base = na + nb - 2.0 * G
        sdiff = sb - sa
        d_nl = jnp.sqrt(jnp.maximum(base + 2.0 * eps * sdiff + deps2, 0.0)) + eps
        d_q = jnp.sqrt(jnp.maximum(base - 2.0 * eps * sdiff + deps2, 0.0))
        return d_nl, d_q

    d_rl, dq_lr = tables(r_ref[:])
    d_ul, dq_lu = tables(u_ref[:])
    erl_ref[...] = jnp.exp(nu_ref[:] - d_rl).astype(jnp.bfloat16)
    eul_ref[...] = jnp.exp(tau_ref[:] - d_ul).astype(jnp.bfloat16)
    dqlr_ref[...] = (nu_ref[:] - dq_lr).astype(jnp.bfloat16)
    dqlu_ref[...] = (tau_ref[:] - dq_lu).astype(jnp.bfloat16)


def _combine_body(rho_ref, erl_ref, eul_ref, dqlr_ref, dqlu_ref,
                  wlr_ref, wlu_ref, out_ref):
    f32 = jnp.float32
    rho_c = rho_ref[:]

    def planes(wref):
        h0 = wref[0:NI, :] + wref[2 * NI:3 * NI, :]
        h1 = wref[NI:2 * NI, :] + wref[3 * NI:4 * NI, :]
        return h0, h1

    wlr0, wlr1 = planes(wlr_ref)
    wlu0, wlu1 = planes(wlu_ref)

    cnt_i = (jnp.sum(wlr0, axis=1, keepdims=True)
             + jnp.sum(wlr1, axis=1, keepdims=True))
    pres_i = (cnt_i > 0.0).astype(f32)
    n_i = jnp.sum(pres_i)
    iota_i = lax.broadcasted_iota(jnp.int32, (NI, 1), 0)
    min_i = jnp.min(jnp.where(cnt_i > 0.0, iota_i, jnp.int32(NI)))
    mi = pres_i + jnp.where(iota_i == min_i, f32(NI) - n_i, 0.0)

    def mult_cols(w0, w1, N):
        cs0 = jnp.sum(w0, axis=0, keepdims=True)
        cs1 = jnp.sum(w1, axis=0, keepdims=True)
        p0 = (cs0 > 0.0).astype(f32)
        p1 = (cs1 > 0.0).astype(f32)
        n = jnp.sum(p0) + jnp.sum(p1)
        io0 = lax.broadcasted_iota(jnp.int32, (1, 128), 1)
        io1 = io0 + 128
        mn = jnp.minimum(jnp.min(jnp.where(cs0 > 0.0, io0, jnp.int32(N))),
                         jnp.min(jnp.where(cs1 > 0.0, io1, jnp.int32(N))))
        extra = f32(N) - n
        m0 = p0 + jnp.where(io0 == mn, extra, 0.0)
        m1 = p1 + jnp.where(io1 == mn, extra, 0.0)
        return m0, m1

    mj0, mj1 = mult_cols(wlr0, wlr1, NJ)
    mk0, mk1 = mult_cols(wlu0, wlu1, NK)

    def wsum(eref, m0, m1):
        e0 = eref[:, 0:128].astype(f32)
        e1 = eref[:, 128:256].astype(f32)
        d = (((1,), (1,)), ((), ()))
        return (lax.dot_general(e0, m0, d, preferred_element_type=f32,
                                precision=lax.Precision.HIGHEST)
                + lax.dot_general(e1, m1, d, preferred_element_type=f32,
                                  precision=lax.Precision.HIGHEST))

    Acol = wsum(erl_ref, mj0, mj1)
    Bcol = jnp.exp(rho_c) * wsum(eul_ref, mk0, mk1)
    z1 = jnp.sum(mi * Acol * Bcol)

    phi_lr = dqlr_ref[:].astype(f32)
    phi_lu = dqlu_ref[:].astype(f32)
    z2 = (jnp.sum(wlr0 * phi_lr[:, 0:128])
          + jnp.sum(wlr1 * phi_lr[:, 128:256])
          + jnp.sum(wlu0 * (rho_c + phi_lu[:, 0:128]))
          + jnp.sum(wlu1 * (rho_c + phi_lu[:, 128:256])))

    out_ref[...] = jnp.broadcast_to(z2 - z1, (1, 1))


def kernel(latent_l, latent_r, latent_u, rho, nu, tau,
           sample_i, sample_j, sample_k, sample_weights):
    si = sample_i.astype(jnp.int32).reshape(NW, ROWS, 128)
    sj = sample_j.astype(jnp.int32).reshape(NW, ROWS, 128)
    sk = sample_k.astype(jnp.int32).reshape(NW, ROWS, 128)
    w = sample_weights.reshape(NW, ROWS, 128)

    wlr2, wlu2 = _hist_call()(_hist_body)(si, sj, sk, w)

    nu_row = nu[:NJ].reshape(1, NJ)
    tau_row = tau[:NK].reshape(1, NK)
    erl, eul, dqlr, dqlu = pl.pallas_call(
        _tables_body,
        out_shape=(jax.ShapeDtypeStruct((NI, NJ), jnp.bfloat16),
                   jax.ShapeDtypeStruct((NI, NK), jnp.bfloat16),
                   jax.ShapeDtypeStruct((NI, NJ), jnp.bfloat16),
                   jax.ShapeDtypeStruct((NI, NK), jnp.bfloat16)),
        grid=(1,),
        in_specs=[
            pl.BlockSpec((NI, D), lambda i: (0, 0)),
            pl.BlockSpec((NJ, D), lambda i: (0, 0)),
            pl.BlockSpec((NK, D), lambda i: (0, 0)),
            pl.BlockSpec((1, NJ), lambda i: (0, 0)),
            pl.BlockSpec((1, NK), lambda i: (0, 0)),
        ],
        out_specs=(pl.BlockSpec((NI, NJ), lambda i: (0, 0)),
                   pl.BlockSpec((NI, NK), lambda i: (0, 0)),
                   pl.BlockSpec((NI, NJ), lambda i: (0, 0)),
                   pl.BlockSpec((NI, NK), lambda i: (0, 0))),
    )(latent_l, latent_r, latent_u, nu_row, tau_row)

    out = pl.pallas_call(
        _combine_body,
        out_shape=jax.ShapeDtypeStruct((1, 1), jnp.float32),
    )(rho[:NI].reshape(NI, 1), erl, eul, dqlr, dqlu,
      wlr2.reshape(NC * 2 * NI, 128), wlu2.reshape(NC * 2 * NI, 128))
    return out[0, 0]
